# baseline (speedup 1.0000x reference)
import jax
import jax.numpy as jnp
from jax import lax
from jax.experimental import pallas as pl
from jax.experimental.pallas import tpu as pltpu
from jax.experimental.pallas import tpu_sc as plsc

N_N = 10000
N_P = 10240
N_E = 320000
D = 128
NC, NS = 2, 16
NW = NC * NS
B = 80
TPW = N_P // NS


NB_H = N_E // NS // B
GRP = 10


def _hist_body(src_ref, dst_ref, hist_hbm, idx_all, ones_v, zero_v, hist_sh,
               sem):
    c = lax.axis_index("c")
    s = lax.axis_index("s")
    one = jnp.full((16,), 1.0, jnp.float32)
    zero = jnp.zeros((16,), jnp.float32)
    for k in range(B // 16):
        ones_v[pl.ds(k * 16, 16)] = one
    for k in range(TPW // 16):
        zero_v[pl.ds(k * 16, 16)] = zero
    pltpu.sync_copy(zero_v, hist_sh.at[pl.ds(s * TPW, TPW)])

    base = s * (N_E // NS)

    def fill(ref):
        def fbody(g, carry):
            for k in range(GRP):
                j = g * GRP + k
                pltpu.async_copy(ref.at[pl.ds(base + j * B, B)],
                                 idx_all.at[j], sem)
            for k in range(GRP):
                pltpu.make_async_copy(ref.at[pl.ds(base + k * B, B)],
                                      idx_all.at[k], sem).wait()
            return carry

        lax.fori_loop(0, NB_H // GRP, fbody, 0)

    @pl.when(c == 0)
    def _():
        fill(src_ref)

    @pl.when(c == 1)
    def _():
        fill(dst_ref)

    plsc.subcore_barrier()

    def body(g, carry):
        for k in range(GRP):
            pltpu.async_copy(ones_v, hist_sh.at[idx_all.at[g * GRP + k]], sem,
                             add=True)
        for k in range(GRP):
            pltpu.make_async_copy(ones_v, hist_sh.at[idx_all.at[g * GRP + k]],
                                  sem).wait()
        return carry

    lax.fori_loop(0, NB_H // GRP, body, 0)
    plsc.subcore_barrier()
    pltpu.sync_copy(hist_sh.at[pl.ds(s * TPW, TPW)],
                    hist_hbm.at[c, 0, pl.ds(s * TPW, TPW)])


B_A = 40
PH = 2
NB_P = N_E // NW // PH // B_A
NBUF = 3


def _agg_body(h_ref, src_ref, dst_ref, part_hbm, sidx_all, didx_all,
              rows0, rows1, rows2,
              acc_sh, gsem0, gsem1, gsem2, ssem0, ssem1, ssem2):
    c = lax.axis_index("c")
    s = lax.axis_index("s")
    wid = s * NC + c
    zero = jnp.zeros((16,), jnp.float32)

    def zbody(j, carry):
        for k in range(D // 16):
            rows0[j, pl.ds(k * 16, 16)] = zero
        return carry

    lax.fori_loop(0, B_A, zbody, 0)

    def zcopy(k, carry):
        pltpu.sync_copy(rows0, acc_sh.at[pl.ds(s * TPW + k * B_A, B_A)])
        return carry

    lax.fori_loop(0, TPW // B_A, zcopy, 0)

    base = wid * (N_E // NW)
    pltpu.sync_copy(src_ref.at[pl.ds(base, N_E // NW)], sidx_all)

    rows = (rows0, rows1, rows2)
    gsem = (gsem0, gsem1, gsem2)
    ssem = (ssem0, ssem1, ssem2)

    GRP_F = 25

    def phase(p, pcarry):
        pbase = base + p * (NB_P * B_A)

        def fbody(g, carry):
            for k in range(GRP_F):
                j = g * GRP_F + k
                pltpu.async_copy(dst_ref.at[pl.ds(pbase + j * B_A, B_A)],
                                 didx_all.at[j], gsem0)
            for k in range(GRP_F):
                pltpu.make_async_copy(dst_ref.at[pl.ds(pbase + k * B_A, B_A)],
                                      didx_all.at[k], gsem0).wait()
            return carry

        lax.fori_loop(0, NB_P // GRP_F, fbody, 0)

        voff = p * (NB_P * B_A)

        for b in range(NBUF):
            pltpu.async_copy(h_ref.at[sidx_all.at[pl.ds(voff + b * B_A, B_A)]],
                             rows[b], gsem[b])

        def body(i, carry):
            for b in range(NBUF):
                j = NBUF * i + b

                @pl.when(j < NB_P)
                def _():
                    pltpu.make_async_copy(
                        h_ref.at[sidx_all.at[pl.ds(voff + j * B_A, B_A)]],
                        rows[b], gsem[b]).wait()
                    pltpu.async_copy(rows[b], acc_sh.at[didx_all.at[j]],
                                     ssem[b], add=True)
                    pltpu.make_async_copy(rows[b], acc_sh.at[didx_all.at[j]],
                                          ssem[b]).wait()

                    @pl.when(j + NBUF < NB_P)
                    def _():
                        pltpu.async_copy(
                            h_ref.at[
                                sidx_all.at[pl.ds(voff + (j + NBUF) * B_A,
                                                  B_A)]],
                            rows[b], gsem[b])

            return carry

        lax.fori_loop(0, (NB_P + NBUF - 1) // NBUF, body, 0)
        return pcarry

    lax.fori_loop(0, PH, phase, 0)
    plsc.subcore_barrier()
    pltpu.sync_copy(acc_sh.at[pl.ds(s * TPW, TPW)],
                    part_hbm.at[c, pl.ds(s * TPW, TPW)])


def _prep_body(feat_ref, od_ref, h_ref):
    h_ref[...] = feat_ref[...] * lax.rsqrt(jnp.maximum(od_ref[...], 1.0))


def _final_body(p0_ref, p1_ref, h_ref, id_ref, w1_ref, w2_ref, o_ref):
    cs = p0_ref[...] + p1_ref[...]
    nd = lax.rsqrt(jnp.maximum(id_ref[...], 1.0))
    acc = jnp.dot(cs, w1_ref[...], preferred_element_type=jnp.float32)
    acc = acc + jnp.dot(h_ref[...] * cs, w2_ref[...],
                        preferred_element_type=jnp.float32)
    o_ref[...] = acc * nd


_mesh = plsc.VectorSubcoreMesh(core_axis_name="c", subcore_axis_name="s")

_hist = pl.kernel(
    _hist_body,
    out_type=jax.ShapeDtypeStruct((2, 1, N_P), jnp.float32),
    mesh=_mesh,
    scratch_types=[
        pltpu.VMEM((NB_H, B), jnp.int32),
        pltpu.VMEM((B,), jnp.float32),
        pltpu.VMEM((TPW,), jnp.float32),
        pltpu.VMEM_SHARED((N_P,), jnp.float32),
        pltpu.SemaphoreType.DMA,
    ],
)

_agg = pl.kernel(
    _agg_body,
    out_type=jax.ShapeDtypeStruct((2, N_P, D), jnp.float32),
    mesh=_mesh,
    scratch_types=[
        pltpu.VMEM((N_E // NW,), jnp.int32),
        pltpu.VMEM((NB_P, B_A), jnp.int32),
        pltpu.VMEM((B_A, D), jnp.float32),
        pltpu.VMEM((B_A, D), jnp.float32),
        pltpu.VMEM((B_A, D), jnp.float32),
        pltpu.VMEM_SHARED((N_P, D), jnp.float32),
        pltpu.SemaphoreType.DMA,
        pltpu.SemaphoreType.DMA,
        pltpu.SemaphoreType.DMA,
        pltpu.SemaphoreType.DMA,
        pltpu.SemaphoreType.DMA,
        pltpu.SemaphoreType.DMA,
    ],
)

_RB = 1000


@jax.jit
def kernel(feat, edge_index, weight1, weight2):
    src = edge_index[0]
    dst = edge_index[1]
    hist = _hist(src, dst)
    od = hist[0, 0, :N_N].reshape(N_N, 1)
    ind = hist[1, 0, :N_N].reshape(N_N, 1)

    h = pl.pallas_call(
        _prep_body,
        grid=(N_N // _RB,),
        in_specs=[
            pl.BlockSpec((_RB, D), lambda i: (i, 0)),
            pl.BlockSpec((_RB, 1), lambda i: (i, 0)),
        ],
        out_specs=pl.BlockSpec((_RB, D), lambda i: (i, 0)),
        out_shape=jax.ShapeDtypeStruct((N_N, D), jnp.float32),
    )(feat, od)

    part = _agg(h, src, dst)

    out = pl.pallas_call(
        _final_body,
        grid=(N_N // _RB,),
        in_specs=[
            pl.BlockSpec((_RB, D), lambda i: (i, 0)),
            pl.BlockSpec((_RB, D), lambda i: (i, 0)),
            pl.BlockSpec((_RB, D), lambda i: (i, 0)),
            pl.BlockSpec((_RB, 1), lambda i: (i, 0)),
            pl.BlockSpec((D, D), lambda i: (0, 0)),
            pl.BlockSpec((D, D), lambda i: (0, 0)),
        ],
        out_specs=pl.BlockSpec((_RB, D), lambda i: (i, 0)),
        out_shape=jax.ShapeDtypeStruct((N_N, D), jnp.float32),
    )(part[0, :N_N], part[1, :N_N], h, ind, weight1, weight2)
    return out


# speedup vs baseline: 1.0130x; 1.0130x over previous
import jax
import jax.numpy as jnp
from jax import lax
from jax.experimental import pallas as pl
from jax.experimental.pallas import tpu as pltpu
from jax.experimental.pallas import tpu_sc as plsc

N_N = 10000
N_P = 10240
N_E = 320000
D = 128
NC, NS = 2, 16
NW = NC * NS
B = 80
TPW = N_P // NS


NB_H = N_E // NS // B
GRP = 10


def _hist_body(src_ref, dst_ref, hist_hbm, idx_all, ones_v, zero_v, hist_sh,
               sem):
    c = lax.axis_index("c")
    s = lax.axis_index("s")
    one = jnp.full((16,), 1.0, jnp.float32)
    zero = jnp.zeros((16,), jnp.float32)
    for k in range(B // 16):
        ones_v[pl.ds(k * 16, 16)] = one
    for k in range(TPW // 16):
        zero_v[pl.ds(k * 16, 16)] = zero
    pltpu.sync_copy(zero_v, hist_sh.at[pl.ds(s * TPW, TPW)])

    base = s * (N_E // NS)

    def fill(ref):
        def fbody(g, carry):
            for k in range(GRP):
                j = g * GRP + k
                pltpu.async_copy(ref.at[pl.ds(base + j * B, B)],
                                 idx_all.at[j], sem)
            for k in range(GRP):
                pltpu.make_async_copy(ref.at[pl.ds(base + k * B, B)],
                                      idx_all.at[k], sem).wait()
            return carry

        lax.fori_loop(0, NB_H // GRP, fbody, 0)

    @pl.when(c == 0)
    def _():
        fill(src_ref)

    @pl.when(c == 1)
    def _():
        fill(dst_ref)

    plsc.subcore_barrier()

    def body(g, carry):
        for k in range(GRP):
            pltpu.async_copy(ones_v, hist_sh.at[idx_all.at[g * GRP + k]], sem,
                             add=True)
        for k in range(GRP):
            pltpu.make_async_copy(ones_v, hist_sh.at[idx_all.at[g * GRP + k]],
                                  sem).wait()
        return carry

    lax.fori_loop(0, NB_H // GRP, body, 0)
    plsc.subcore_barrier()
    pltpu.sync_copy(hist_sh.at[pl.ds(s * TPW, TPW)],
                    hist_hbm.at[c, 0, pl.ds(s * TPW, TPW)])


B_A = 80
NB_A = N_E // NW // B_A
NBUF = 2


def _agg_body(h_ref, src_ref, dst_ref, part_hbm, sidx_all, didx_all,
              rows0, rows1,
              acc_sh, gsem0, gsem1, ssem0, ssem1):
    c = lax.axis_index("c")
    s = lax.axis_index("s")
    wid = s * NC + c
    zero = jnp.zeros((16,), jnp.float32)

    def zbody(j, carry):
        for k in range(D // 16):
            rows0[j, pl.ds(k * 16, 16)] = zero
        return carry

    lax.fori_loop(0, B_A, zbody, 0)

    def zcopy(k, carry):
        pltpu.sync_copy(rows0, acc_sh.at[pl.ds(s * TPW + k * B_A, B_A)])
        return carry

    lax.fori_loop(0, TPW // B_A, zcopy, 0)

    base = wid * (N_E // NW)
    pltpu.sync_copy(src_ref.at[pl.ds(base, N_E // NW)], sidx_all)

    GRP_F = 25

    def fbody(g, carry):
        for k in range(GRP_F):
            j = g * GRP_F + k
            pltpu.async_copy(dst_ref.at[pl.ds(base + j * B_A, B_A)],
                             didx_all.at[j], gsem0)
        for k in range(GRP_F):
            pltpu.make_async_copy(dst_ref.at[pl.ds(base + k * B_A, B_A)],
                                  didx_all.at[k], gsem0).wait()
        return carry

    lax.fori_loop(0, NB_A // GRP_F, fbody, 0)
    plsc.subcore_barrier()

    rows = (rows0, rows1)
    gsem = (gsem0, gsem1)
    ssem = (ssem0, ssem1)

    for b in range(NBUF):
        pltpu.async_copy(h_ref.at[sidx_all.at[pl.ds(b * B_A, B_A)]], rows[b],
                         gsem[b])

    def body(i, carry):
        for b in range(NBUF):
            j = NBUF * i + b

            @pl.when(j < NB_A)
            def _():
                pltpu.make_async_copy(
                    h_ref.at[sidx_all.at[pl.ds(j * B_A, B_A)]],
                    rows[b], gsem[b]).wait()
                pltpu.async_copy(rows[b], acc_sh.at[didx_all.at[j]],
                                 ssem[b], add=True)
                pltpu.make_async_copy(rows[b], acc_sh.at[didx_all.at[j]],
                                      ssem[b]).wait()

                @pl.when(j + NBUF < NB_A)
                def _():
                    pltpu.async_copy(
                        h_ref.at[sidx_all.at[pl.ds((j + NBUF) * B_A, B_A)]],
                        rows[b], gsem[b])

        return carry

    lax.fori_loop(0, (NB_A + NBUF - 1) // NBUF, body, 0)
    plsc.subcore_barrier()
    pltpu.sync_copy(acc_sh.at[pl.ds(s * TPW, TPW)],
                    part_hbm.at[c, pl.ds(s * TPW, TPW)])


def _prep_body(feat_ref, od_ref, h_ref):
    h_ref[...] = feat_ref[...] * lax.rsqrt(jnp.maximum(od_ref[...], 1.0))


def _final_body(p0_ref, p1_ref, h_ref, id_ref, w1_ref, w2_ref, o_ref):
    cs = p0_ref[...] + p1_ref[...]
    nd = lax.rsqrt(jnp.maximum(id_ref[...], 1.0))
    acc = jnp.dot(cs, w1_ref[...], preferred_element_type=jnp.float32)
    acc = acc + jnp.dot(h_ref[...] * cs, w2_ref[...],
                        preferred_element_type=jnp.float32)
    o_ref[...] = acc * nd


_mesh = plsc.VectorSubcoreMesh(core_axis_name="c", subcore_axis_name="s")

_hist = pl.kernel(
    _hist_body,
    out_type=jax.ShapeDtypeStruct((2, 1, N_P), jnp.float32),
    mesh=_mesh,
    scratch_types=[
        pltpu.VMEM((NB_H, B), jnp.int32),
        pltpu.VMEM((B,), jnp.float32),
        pltpu.VMEM((TPW,), jnp.float32),
        pltpu.VMEM_SHARED((N_P,), jnp.float32),
        pltpu.SemaphoreType.DMA,
    ],
)

_agg = pl.kernel(
    _agg_body,
    out_type=jax.ShapeDtypeStruct((2, N_P, D), jnp.float32),
    mesh=_mesh,
    scratch_types=[
        pltpu.VMEM((N_E // NW,), jnp.int32),
        pltpu.VMEM((NB_A, B_A), jnp.int32),
        pltpu.VMEM((B_A, D), jnp.float32),
        pltpu.VMEM((B_A, D), jnp.float32),
        pltpu.VMEM_SHARED((N_P, D), jnp.float32),
        pltpu.SemaphoreType.DMA,
        pltpu.SemaphoreType.DMA,
        pltpu.SemaphoreType.DMA,
        pltpu.SemaphoreType.DMA,
    ],
)

_RB = 1000


@jax.jit
def kernel(feat, edge_index, weight1, weight2):
    src = edge_index[0]
    dst = edge_index[1]
    hist = _hist(src, dst)
    od = hist[0, 0, :N_N].reshape(N_N, 1)
    ind = hist[1, 0, :N_N].reshape(N_N, 1)

    h = pl.pallas_call(
        _prep_body,
        grid=(N_N // _RB,),
        in_specs=[
            pl.BlockSpec((_RB, D), lambda i: (i, 0)),
            pl.BlockSpec((_RB, 1), lambda i: (i, 0)),
        ],
        out_specs=pl.BlockSpec((_RB, D), lambda i: (i, 0)),
        out_shape=jax.ShapeDtypeStruct((N_N, D), jnp.float32),
    )(feat, od)

    part = _agg(h, src, dst)

    out = pl.pallas_call(
        _final_body,
        grid=(N_N // _RB,),
        in_specs=[
            pl.BlockSpec((_RB, D), lambda i: (i, 0)),
            pl.BlockSpec((_RB, D), lambda i: (i, 0)),
            pl.BlockSpec((_RB, D), lambda i: (i, 0)),
            pl.BlockSpec((_RB, 1), lambda i: (i, 0)),
            pl.BlockSpec((D, D), lambda i: (0, 0)),
            pl.BlockSpec((D, D), lambda i: (0, 0)),
        ],
        out_specs=pl.BlockSpec((_RB, D), lambda i: (i, 0)),
        out_shape=jax.ShapeDtypeStruct((N_N, D), jnp.float32),
    )(part[0, :N_N], part[1, :N_N], h, ind, weight1, weight2)
    return out


# speedup vs baseline: 1.1035x; 1.0893x over previous
import jax
import jax.numpy as jnp
from jax import lax
from jax.experimental import pallas as pl
from jax.experimental.pallas import tpu as pltpu
from jax.experimental.pallas import tpu_sc as plsc

N_N = 10000
N_P = 10240
N_E = 320000
D = 128
NC, NS = 2, 16
NW = NC * NS
B = 80
TPW = N_P // NS


NB_H = N_E // NS // B
GRP = 10


def _hist_body(ei_ref, hist_hbm, idx_all, ones_v, zero_v, hist_sh,
               sem):
    c = lax.axis_index("c")
    s = lax.axis_index("s")
    one = jnp.full((16,), 1.0, jnp.float32)
    zero = jnp.zeros((16,), jnp.float32)
    for k in range(B // 16):
        ones_v[pl.ds(k * 16, 16)] = one
    for k in range(TPW // 16):
        zero_v[pl.ds(k * 16, 16)] = zero
    pltpu.sync_copy(zero_v, hist_sh.at[pl.ds(s * TPW, TPW)])

    base = c * N_E + s * (N_E // NS)

    def fbody(g, carry):
        for k in range(GRP):
            j = g * GRP + k
            pltpu.async_copy(ei_ref.at[pl.ds(base + j * B, B)],
                             idx_all.at[j], sem)
        for k in range(GRP):
            pltpu.make_async_copy(ei_ref.at[pl.ds(base + k * B, B)],
                                  idx_all.at[k], sem).wait()
        return carry

    lax.fori_loop(0, NB_H // GRP, fbody, 0)
    plsc.subcore_barrier()

    def body(g, carry):
        for k in range(GRP):
            pltpu.async_copy(ones_v, hist_sh.at[idx_all.at[g * GRP + k]], sem,
                             add=True)
        for k in range(GRP):
            pltpu.make_async_copy(ones_v, hist_sh.at[idx_all.at[g * GRP + k]],
                                  sem).wait()
        return carry

    lax.fori_loop(0, NB_H // GRP, body, 0)
    plsc.subcore_barrier()
    pltpu.sync_copy(hist_sh.at[pl.ds(s * TPW, TPW)],
                    hist_hbm.at[c, 0, pl.ds(s * TPW, TPW)])


B_A = 80
NB_A = N_E // NW // B_A
NBUF = 2


def _agg_body(h_ref, ei_ref, part_hbm, sidx_all, didx_all,
              rows0, rows1,
              acc_sh, gsem0, gsem1, ssem0, ssem1):
    c = lax.axis_index("c")
    s = lax.axis_index("s")
    wid = s * NC + c
    zero = jnp.zeros((16,), jnp.float32)

    def zbody(j, carry):
        for k in range(D // 16):
            rows0[j, pl.ds(k * 16, 16)] = zero
        return carry

    lax.fori_loop(0, B_A, zbody, 0)

    def zcopy(k, carry):
        pltpu.sync_copy(rows0, acc_sh.at[pl.ds(s * TPW + k * B_A, B_A)])
        return carry

    nz = jnp.where(s == NS - 1, (N_N - (NS - 1) * TPW) // B_A, TPW // B_A)
    lax.fori_loop(0, nz, zcopy, 0)

    base = wid * (N_E // NW)
    pltpu.sync_copy(ei_ref.at[pl.ds(base, N_E // NW)], sidx_all)

    GRP_F = 25

    def fbody(g, carry):
        for k in range(GRP_F):
            j = g * GRP_F + k
            pltpu.async_copy(ei_ref.at[pl.ds(N_E + base + j * B_A, B_A)],
                             didx_all.at[j], gsem0)
        for k in range(GRP_F):
            pltpu.make_async_copy(ei_ref.at[pl.ds(N_E + base + k * B_A, B_A)],
                                  didx_all.at[k], gsem0).wait()
        return carry

    lax.fori_loop(0, NB_A // GRP_F, fbody, 0)
    plsc.subcore_barrier()

    rows = (rows0, rows1)
    gsem = (gsem0, gsem1)
    ssem = (ssem0, ssem1)

    for b in range(NBUF):
        pltpu.async_copy(h_ref.at[sidx_all.at[pl.ds(b * B_A, B_A)]], rows[b],
                         gsem[b])

    def body(i, carry):
        for b in range(NBUF):
            j = NBUF * i + b

            @pl.when(j < NB_A)
            def _():
                pltpu.make_async_copy(
                    h_ref.at[sidx_all.at[pl.ds(j * B_A, B_A)]],
                    rows[b], gsem[b]).wait()
                pltpu.async_copy(rows[b], acc_sh.at[didx_all.at[j]],
                                 ssem[b], add=True)
                pltpu.make_async_copy(rows[b], acc_sh.at[didx_all.at[j]],
                                      ssem[b]).wait()

                @pl.when(j + NBUF < NB_A)
                def _():
                    pltpu.async_copy(
                        h_ref.at[sidx_all.at[pl.ds((j + NBUF) * B_A, B_A)]],
                        rows[b], gsem[b])

        return carry

    lax.fori_loop(0, (NB_A + NBUF - 1) // NBUF, body, 0)
    plsc.subcore_barrier()
    LAST = N_N - (NS - 1) * TPW

    @pl.when(s < NS - 1)
    def _():
        pltpu.sync_copy(acc_sh.at[pl.ds(s * TPW, TPW)],
                        part_hbm.at[c, pl.ds(s * TPW, TPW)])

    @pl.when(s == NS - 1)
    def _():
        pltpu.sync_copy(acc_sh.at[pl.ds(s * TPW, LAST)],
                        part_hbm.at[c, pl.ds(s * TPW, LAST)])


def _prep_body(feat_ref, od_ref, h_ref):
    h_ref[...] = feat_ref[...] * lax.rsqrt(jnp.maximum(od_ref[...], 1.0))


def _final_body(p0_ref, p1_ref, h_ref, id_ref, w1_ref, w2_ref, o_ref):
    cs = p0_ref[0] + p1_ref[0]
    nd = lax.rsqrt(jnp.maximum(id_ref[...], 1.0))
    acc = jnp.dot(cs, w1_ref[...], preferred_element_type=jnp.float32)
    acc = acc + jnp.dot(h_ref[...] * cs, w2_ref[...],
                        preferred_element_type=jnp.float32)
    o_ref[...] = acc * nd


_mesh = plsc.VectorSubcoreMesh(core_axis_name="c", subcore_axis_name="s")

_hist = pl.kernel(
    _hist_body,
    out_type=jax.ShapeDtypeStruct((2, 1, N_P), jnp.float32),
    mesh=_mesh,
    scratch_types=[
        pltpu.VMEM((NB_H, B), jnp.int32),
        pltpu.VMEM((B,), jnp.float32),
        pltpu.VMEM((TPW,), jnp.float32),
        pltpu.VMEM_SHARED((N_P,), jnp.float32),
        pltpu.SemaphoreType.DMA,
    ],
)

_agg = pl.kernel(
    _agg_body,
    out_type=jax.ShapeDtypeStruct((2, N_N, D), jnp.float32),
    mesh=_mesh,
    scratch_types=[
        pltpu.VMEM((N_E // NW,), jnp.int32),
        pltpu.VMEM((NB_A, B_A), jnp.int32),
        pltpu.VMEM((B_A, D), jnp.float32),
        pltpu.VMEM((B_A, D), jnp.float32),
        pltpu.VMEM_SHARED((N_N, D), jnp.float32),
        pltpu.SemaphoreType.DMA,
        pltpu.SemaphoreType.DMA,
        pltpu.SemaphoreType.DMA,
        pltpu.SemaphoreType.DMA,
    ],
)

_RB = 1000


@jax.jit
def kernel(feat, edge_index, weight1, weight2):
    ei_flat = edge_index.reshape(2 * N_E)
    hist = _hist(ei_flat)
    od = hist[0, 0, :N_N].reshape(N_N, 1)
    ind = hist[1, 0, :N_N].reshape(N_N, 1)

    h = pl.pallas_call(
        _prep_body,
        grid=(N_N // _RB,),
        in_specs=[
            pl.BlockSpec((_RB, D), lambda i: (i, 0)),
            pl.BlockSpec((_RB, 1), lambda i: (i, 0)),
        ],
        out_specs=pl.BlockSpec((_RB, D), lambda i: (i, 0)),
        out_shape=jax.ShapeDtypeStruct((N_N, D), jnp.float32),
    )(feat, od)

    part = _agg(h, ei_flat)

    out = pl.pallas_call(
        _final_body,
        grid=(N_N // _RB,),
        in_specs=[
            pl.BlockSpec((1, _RB, D), lambda i: (0, i, 0)),
            pl.BlockSpec((1, _RB, D), lambda i: (1, i, 0)),
            pl.BlockSpec((_RB, D), lambda i: (i, 0)),
            pl.BlockSpec((_RB, 1), lambda i: (i, 0)),
            pl.BlockSpec((D, D), lambda i: (0, 0)),
            pl.BlockSpec((D, D), lambda i: (0, 0)),
        ],
        out_specs=pl.BlockSpec((_RB, D), lambda i: (i, 0)),
        out_shape=jax.ShapeDtypeStruct((N_N, D), jnp.float32),
    )(part, part, h, ind, weight1, weight2)
    return out


# speedup vs baseline: 1.1475x; 1.0398x over previous
import jax
import jax.numpy as jnp
from jax import lax
from jax.experimental import pallas as pl
from jax.experimental.pallas import tpu as pltpu
from jax.experimental.pallas import tpu_sc as plsc

N_N = 10000
N_P = 10240
N_E = 320000
D = 128
NC, NS = 2, 16
NW = NC * NS
B = 80
TPW = N_P // NS


NB_H = N_E // NW // B
GRP = 25


def _hist_body(ei_ref, hist_hbm, idx_all, ones_v, zero_v, hist_sh,
               sem):
    c = lax.axis_index("c")
    s = lax.axis_index("s")
    one = jnp.full((16,), 1.0, jnp.float32)
    zero = jnp.zeros((16,), jnp.float32)
    for k in range(B // 16):
        ones_v[pl.ds(k * 16, 16)] = one
    for k in range(TPW // 16):
        zero_v[pl.ds(k * 16, 16)] = zero
    pltpu.sync_copy(zero_v, hist_sh.at[pl.ds(s * TPW, TPW)])

    base = (s * NC + c) * (N_E // NW)

    def fbody(g, carry):
        for k in range(GRP):
            j = g * GRP + k
            pltpu.async_copy(ei_ref.at[pl.ds(base + j * B, B)],
                             idx_all.at[j], sem)
        for k in range(GRP):
            pltpu.make_async_copy(ei_ref.at[pl.ds(base + k * B, B)],
                                  idx_all.at[k], sem).wait()
        return carry

    lax.fori_loop(0, NB_H // GRP, fbody, 0)
    plsc.subcore_barrier()

    def body(g, carry):
        for k in range(GRP):
            pltpu.async_copy(ones_v, hist_sh.at[idx_all.at[g * GRP + k]], sem,
                             add=True)
        for k in range(GRP):
            pltpu.make_async_copy(ones_v, hist_sh.at[idx_all.at[g * GRP + k]],
                                  sem).wait()
        return carry

    lax.fori_loop(0, NB_H // GRP, body, 0)
    plsc.subcore_barrier()
    pltpu.sync_copy(hist_sh.at[pl.ds(s * TPW, TPW)],
                    hist_hbm.at[c, 0, pl.ds(s * TPW, TPW)])


B_A = 80
NB_A = N_E // NW // B_A
NBUF = 2


def _agg_body(h_ref, ei_ref, part_hbm, ideg_hbm, sidx_all, didx_all,
              rows0, rows1, ones_v, zid_v,
              acc_sh, ideg_sh, gsem0, gsem1, ssem0, ssem1, hsem):
    c = lax.axis_index("c")
    s = lax.axis_index("s")
    wid = s * NC + c
    zero = jnp.zeros((16,), jnp.float32)

    def zbody(j, carry):
        for k in range(D // 16):
            rows0[j, pl.ds(k * 16, 16)] = zero
        return carry

    lax.fori_loop(0, B_A, zbody, 0)
    one = jnp.full((16,), 1.0, jnp.float32)
    zero16 = jnp.zeros((16,), jnp.float32)
    for k in range(B_A // 16):
        ones_v[pl.ds(k * 16, 16)] = one
    for k in range(TPW // 16):
        zid_v[pl.ds(k * 16, 16)] = zero16
    pltpu.sync_copy(zid_v, ideg_sh.at[pl.ds(s * TPW, TPW)])

    def zcopy(k, carry):
        pltpu.sync_copy(rows0, acc_sh.at[pl.ds(s * TPW + k * B_A, B_A)])
        return carry

    nz = jnp.where(s == NS - 1, (N_N - (NS - 1) * TPW) // B_A, TPW // B_A)
    lax.fori_loop(0, nz, zcopy, 0)

    base = wid * (N_E // NW)
    pltpu.sync_copy(ei_ref.at[pl.ds(base, N_E // NW)], sidx_all)

    GRP_F = 25

    def fbody(g, carry):
        for k in range(GRP_F):
            j = g * GRP_F + k
            pltpu.async_copy(ei_ref.at[pl.ds(N_E + base + j * B_A, B_A)],
                             didx_all.at[j], gsem0)
        for k in range(GRP_F):
            pltpu.make_async_copy(ei_ref.at[pl.ds(N_E + base + k * B_A, B_A)],
                                  didx_all.at[k], gsem0).wait()
        return carry

    lax.fori_loop(0, NB_A // GRP_F, fbody, 0)
    plsc.subcore_barrier()

    rows = (rows0, rows1)
    gsem = (gsem0, gsem1)
    ssem = (ssem0, ssem1)

    for b in range(NBUF):
        pltpu.async_copy(h_ref.at[sidx_all.at[pl.ds(b * B_A, B_A)]], rows[b],
                         gsem[b])

    def body(i, carry):
        for b in range(NBUF):
            j = NBUF * i + b

            @pl.when(j < NB_A)
            def _():
                pltpu.make_async_copy(
                    h_ref.at[sidx_all.at[pl.ds(j * B_A, B_A)]],
                    rows[b], gsem[b]).wait()
                pltpu.async_copy(rows[b], acc_sh.at[didx_all.at[j]],
                                 ssem[b], add=True)
                pltpu.async_copy(ones_v, ideg_sh.at[didx_all.at[j]],
                                 hsem, add=True)
                pltpu.make_async_copy(rows[b], acc_sh.at[didx_all.at[j]],
                                      ssem[b]).wait()

                @pl.when(j + NBUF < NB_A)
                def _():
                    pltpu.async_copy(
                        h_ref.at[sidx_all.at[pl.ds((j + NBUF) * B_A, B_A)]],
                        rows[b], gsem[b])

        return carry

    lax.fori_loop(0, (NB_A + NBUF - 1) // NBUF, body, 0)

    def hdrain(j, carry):
        pltpu.make_async_copy(ones_v, ideg_sh.at[didx_all.at[j]],
                              hsem).wait()
        return carry

    lax.fori_loop(0, NB_A, hdrain, 0)
    plsc.subcore_barrier()
    pltpu.sync_copy(ideg_sh.at[pl.ds(s * TPW, TPW)],
                    ideg_hbm.at[c, 0, pl.ds(s * TPW, TPW)])
    LAST = N_N - (NS - 1) * TPW

    @pl.when(s < NS - 1)
    def _():
        pltpu.sync_copy(acc_sh.at[pl.ds(s * TPW, TPW)],
                        part_hbm.at[c, pl.ds(s * TPW, TPW)])

    @pl.when(s == NS - 1)
    def _():
        pltpu.sync_copy(acc_sh.at[pl.ds(s * TPW, LAST)],
                        part_hbm.at[c, pl.ds(s * TPW, LAST)])


def _prep_body(feat_ref, od_ref, h_ref):
    h_ref[...] = feat_ref[...] * lax.rsqrt(jnp.maximum(od_ref[...], 1.0))


def _final_body(p0_ref, p1_ref, h_ref, id_ref, w1_ref, w2_ref, o_ref):
    cs = p0_ref[0] + p1_ref[0]
    nd = lax.rsqrt(jnp.maximum(id_ref[...], 1.0))
    acc = jnp.dot(cs, w1_ref[...], preferred_element_type=jnp.float32)
    acc = acc + jnp.dot(h_ref[...] * cs, w2_ref[...],
                        preferred_element_type=jnp.float32)
    o_ref[...] = acc * nd


_mesh = plsc.VectorSubcoreMesh(core_axis_name="c", subcore_axis_name="s")

_hist = pl.kernel(
    _hist_body,
    out_type=jax.ShapeDtypeStruct((2, 1, N_P), jnp.float32),
    mesh=_mesh,
    scratch_types=[
        pltpu.VMEM((NB_H, B), jnp.int32),
        pltpu.VMEM((B,), jnp.float32),
        pltpu.VMEM((TPW,), jnp.float32),
        pltpu.VMEM_SHARED((N_P,), jnp.float32),
        pltpu.SemaphoreType.DMA,
    ],
)

_agg = pl.kernel(
    _agg_body,
    out_type=[jax.ShapeDtypeStruct((2, N_N, D), jnp.float32),
              jax.ShapeDtypeStruct((2, 1, N_P), jnp.float32)],
    mesh=_mesh,
    scratch_types=[
        pltpu.VMEM((N_E // NW,), jnp.int32),
        pltpu.VMEM((NB_A, B_A), jnp.int32),
        pltpu.VMEM((B_A, D), jnp.float32),
        pltpu.VMEM((B_A, D), jnp.float32),
        pltpu.VMEM((B_A,), jnp.float32),
        pltpu.VMEM((TPW,), jnp.float32),
        pltpu.VMEM_SHARED((N_N, D), jnp.float32),
        pltpu.VMEM_SHARED((N_P,), jnp.float32),
        pltpu.SemaphoreType.DMA,
        pltpu.SemaphoreType.DMA,
        pltpu.SemaphoreType.DMA,
        pltpu.SemaphoreType.DMA,
        pltpu.SemaphoreType.DMA,
    ],
)

_RB = 1000


@jax.jit
def kernel(feat, edge_index, weight1, weight2):
    ei_flat = edge_index.reshape(2 * N_E)
    hist = _hist(ei_flat)
    od = (hist[0, 0, :N_N] + hist[1, 0, :N_N]).reshape(N_N, 1)

    h = pl.pallas_call(
        _prep_body,
        grid=(N_N // _RB,),
        in_specs=[
            pl.BlockSpec((_RB, D), lambda i: (i, 0)),
            pl.BlockSpec((_RB, 1), lambda i: (i, 0)),
        ],
        out_specs=pl.BlockSpec((_RB, D), lambda i: (i, 0)),
        out_shape=jax.ShapeDtypeStruct((N_N, D), jnp.float32),
    )(feat, od)

    part, ideg = _agg(h, ei_flat)
    ind = (ideg[0, 0, :N_N] + ideg[1, 0, :N_N]).reshape(N_N, 1)

    out = pl.pallas_call(
        _final_body,
        grid=(N_N // _RB,),
        in_specs=[
            pl.BlockSpec((1, _RB, D), lambda i: (0, i, 0)),
            pl.BlockSpec((1, _RB, D), lambda i: (1, i, 0)),
            pl.BlockSpec((_RB, D), lambda i: (i, 0)),
            pl.BlockSpec((_RB, 1), lambda i: (i, 0)),
            pl.BlockSpec((D, D), lambda i: (0, 0)),
            pl.BlockSpec((D, D), lambda i: (0, 0)),
        ],
        out_specs=pl.BlockSpec((_RB, D), lambda i: (i, 0)),
        out_shape=jax.ShapeDtypeStruct((N_N, D), jnp.float32),
    )(part, part, h, ind, weight1, weight2)
    return out


# speedup vs baseline: 1.1667x; 1.0168x over previous
import jax
import jax.numpy as jnp
from jax import lax
from jax.experimental import pallas as pl
from jax.experimental.pallas import tpu as pltpu
from jax.experimental.pallas import tpu_sc as plsc

N_N = 10000
N_P = 10240
N_E = 320000
D = 128
NC, NS = 2, 16
NW = NC * NS
B = 80
TPW = N_P // NS


NB_H = N_E // NW // B
GRP = 25


def _hist_body(ei_ref, hist_hbm, idx_all, ones_v, zero_v, hist_sh,
               sem):
    c = lax.axis_index("c")
    s = lax.axis_index("s")
    one = jnp.full((16,), 1.0, jnp.float32)
    zero = jnp.zeros((16,), jnp.float32)
    for k in range(B // 16):
        ones_v[pl.ds(k * 16, 16)] = one
    for k in range(TPW // 16):
        zero_v[pl.ds(k * 16, 16)] = zero
    pltpu.sync_copy(zero_v, hist_sh.at[pl.ds(s * TPW, TPW)])

    base = (s * NC + c) * (N_E // NW)

    def fbody(g, carry):
        for k in range(GRP):
            j = g * GRP + k
            pltpu.async_copy(ei_ref.at[pl.ds(base + j * B, B)],
                             idx_all.at[j], sem)
        for k in range(GRP):
            pltpu.make_async_copy(ei_ref.at[pl.ds(base + k * B, B)],
                                  idx_all.at[k], sem).wait()
        return carry

    lax.fori_loop(0, NB_H // GRP, fbody, 0)
    plsc.subcore_barrier()

    def body(g, carry):
        for k in range(GRP):
            pltpu.async_copy(ones_v, hist_sh.at[idx_all.at[g * GRP + k]], sem,
                             add=True)
        for k in range(GRP):
            pltpu.make_async_copy(ones_v, hist_sh.at[idx_all.at[g * GRP + k]],
                                  sem).wait()
        return carry

    lax.fori_loop(0, NB_H // GRP, body, 0)
    plsc.subcore_barrier()
    pltpu.sync_copy(hist_sh.at[pl.ds(s * TPW, TPW)],
                    hist_hbm.at[c, 0, pl.ds(s * TPW, TPW)])


B_A = 80
NB_A = N_E // NW // B_A
NBUF = 2


def _agg_body(h_ref, ei_ref, part_hbm, ideg_hbm, sidx_all, didx_all,
              rows0, rows1, ones_v, zid_v,
              acc_sh, ideg_sh, gsem0, gsem1, ssem0, ssem1, hsem):
    c = lax.axis_index("c")
    s = lax.axis_index("s")
    wid = s * NC + c
    zero = jnp.zeros((16,), jnp.float32)

    def zbody(j, carry):
        for k in range(D // 16):
            rows0[j, pl.ds(k * 16, 16)] = zero
        return carry

    lax.fori_loop(0, B_A, zbody, 0)
    one = jnp.full((16,), 1.0, jnp.float32)
    zero16 = jnp.zeros((16,), jnp.float32)
    for k in range(B_A // 16):
        ones_v[pl.ds(k * 16, 16)] = one
    for k in range(TPW // 16):
        zid_v[pl.ds(k * 16, 16)] = zero16

    nz = jnp.where(s == NS - 1, (N_N - (NS - 1) * TPW) // B_A, TPW // B_A)

    def zfire(k, carry):
        pltpu.async_copy(rows0, acc_sh.at[pl.ds(s * TPW + k * B_A, B_A)],
                         gsem1)
        return carry

    lax.fori_loop(0, nz, zfire, 0)
    pltpu.async_copy(zid_v, ideg_sh.at[pl.ds(s * TPW, TPW)], gsem1)

    base = wid * (N_E // NW)
    pltpu.sync_copy(ei_ref.at[pl.ds(base, N_E // NW)], sidx_all)

    GRP_F = 25

    def fbody(g, carry):
        for k in range(GRP_F):
            j = g * GRP_F + k
            pltpu.async_copy(ei_ref.at[pl.ds(N_E + base + j * B_A, B_A)],
                             didx_all.at[j], gsem0)
        for k in range(GRP_F):
            pltpu.make_async_copy(ei_ref.at[pl.ds(N_E + base + k * B_A, B_A)],
                                  didx_all.at[k], gsem0).wait()
        return carry

    lax.fori_loop(0, NB_A // GRP_F, fbody, 0)

    def zdrain(k, carry):
        pltpu.make_async_copy(rows0, acc_sh.at[pl.ds(s * TPW + k * B_A, B_A)],
                              gsem1).wait()
        return carry

    lax.fori_loop(0, nz, zdrain, 0)
    pltpu.make_async_copy(zid_v, ideg_sh.at[pl.ds(s * TPW, TPW)],
                          gsem1).wait()
    plsc.subcore_barrier()

    rows = (rows0, rows1)
    gsem = (gsem0, gsem1)
    ssem = (ssem0, ssem1)

    for b in range(NBUF):
        pltpu.async_copy(h_ref.at[sidx_all.at[pl.ds(b * B_A, B_A)]], rows[b],
                         gsem[b])

    def body(i, carry):
        for b in range(NBUF):
            j = NBUF * i + b

            @pl.when(j < NB_A)
            def _():
                pltpu.make_async_copy(
                    h_ref.at[sidx_all.at[pl.ds(j * B_A, B_A)]],
                    rows[b], gsem[b]).wait()
                pltpu.async_copy(rows[b], acc_sh.at[didx_all.at[j]],
                                 ssem[b], add=True)
                pltpu.async_copy(ones_v, ideg_sh.at[didx_all.at[j]],
                                 hsem, add=True)
                pltpu.make_async_copy(rows[b], acc_sh.at[didx_all.at[j]],
                                      ssem[b]).wait()

                @pl.when(j + NBUF < NB_A)
                def _():
                    pltpu.async_copy(
                        h_ref.at[sidx_all.at[pl.ds((j + NBUF) * B_A, B_A)]],
                        rows[b], gsem[b])

        return carry

    lax.fori_loop(0, (NB_A + NBUF - 1) // NBUF, body, 0)

    def hdrain(j, carry):
        pltpu.make_async_copy(ones_v, ideg_sh.at[didx_all.at[j]],
                              hsem).wait()
        return carry

    lax.fori_loop(0, NB_A, hdrain, 0)
    plsc.subcore_barrier()
    pltpu.sync_copy(ideg_sh.at[pl.ds(s * TPW, TPW)],
                    ideg_hbm.at[c, 0, pl.ds(s * TPW, TPW)])
    LAST = N_N - (NS - 1) * TPW

    @pl.when(s < NS - 1)
    def _():
        pltpu.sync_copy(acc_sh.at[pl.ds(s * TPW, TPW)],
                        part_hbm.at[c, pl.ds(s * TPW, TPW)])

    @pl.when(s == NS - 1)
    def _():
        pltpu.sync_copy(acc_sh.at[pl.ds(s * TPW, LAST)],
                        part_hbm.at[c, pl.ds(s * TPW, LAST)])


def _prep_body(feat_ref, od_ref, h_ref):
    h_ref[...] = feat_ref[...] * lax.rsqrt(jnp.maximum(od_ref[...], 1.0))


def _final_body(p0_ref, p1_ref, h_ref, id_ref, w1_ref, w2_ref, o_ref):
    cs = p0_ref[0] + p1_ref[0]
    nd = lax.rsqrt(jnp.maximum(id_ref[...], 1.0))
    acc = jnp.dot(cs, w1_ref[...], preferred_element_type=jnp.float32)
    acc = acc + jnp.dot(h_ref[...] * cs, w2_ref[...],
                        preferred_element_type=jnp.float32)
    o_ref[...] = acc * nd


_mesh = plsc.VectorSubcoreMesh(core_axis_name="c", subcore_axis_name="s")

_hist = pl.kernel(
    _hist_body,
    out_type=jax.ShapeDtypeStruct((2, 1, N_P), jnp.float32),
    mesh=_mesh,
    scratch_types=[
        pltpu.VMEM((NB_H, B), jnp.int32),
        pltpu.VMEM((B,), jnp.float32),
        pltpu.VMEM((TPW,), jnp.float32),
        pltpu.VMEM_SHARED((N_P,), jnp.float32),
        pltpu.SemaphoreType.DMA,
    ],
)

_agg = pl.kernel(
    _agg_body,
    out_type=[jax.ShapeDtypeStruct((2, N_N, D), jnp.float32),
              jax.ShapeDtypeStruct((2, 1, N_P), jnp.float32)],
    mesh=_mesh,
    scratch_types=[
        pltpu.VMEM((N_E // NW,), jnp.int32),
        pltpu.VMEM((NB_A, B_A), jnp.int32),
        pltpu.VMEM((B_A, D), jnp.float32),
        pltpu.VMEM((B_A, D), jnp.float32),
        pltpu.VMEM((B_A,), jnp.float32),
        pltpu.VMEM((TPW,), jnp.float32),
        pltpu.VMEM_SHARED((N_N, D), jnp.float32),
        pltpu.VMEM_SHARED((N_P,), jnp.float32),
        pltpu.SemaphoreType.DMA,
        pltpu.SemaphoreType.DMA,
        pltpu.SemaphoreType.DMA,
        pltpu.SemaphoreType.DMA,
        pltpu.SemaphoreType.DMA,
    ],
)

_RB = 1000


@jax.jit
def kernel(feat, edge_index, weight1, weight2):
    ei_flat = edge_index.reshape(2 * N_E)
    hist = _hist(ei_flat)
    od = (hist[0, 0, :N_N] + hist[1, 0, :N_N]).reshape(N_N, 1)

    h = pl.pallas_call(
        _prep_body,
        grid=(N_N // _RB,),
        in_specs=[
            pl.BlockSpec((_RB, D), lambda i: (i, 0)),
            pl.BlockSpec((_RB, 1), lambda i: (i, 0)),
        ],
        out_specs=pl.BlockSpec((_RB, D), lambda i: (i, 0)),
        out_shape=jax.ShapeDtypeStruct((N_N, D), jnp.float32),
    )(feat, od)

    part, ideg = _agg(h, ei_flat)
    ind = (ideg[0, 0, :N_N] + ideg[1, 0, :N_N]).reshape(N_N, 1)

    out = pl.pallas_call(
        _final_body,
        grid=(N_N // _RB,),
        in_specs=[
            pl.BlockSpec((1, _RB, D), lambda i: (0, i, 0)),
            pl.BlockSpec((1, _RB, D), lambda i: (1, i, 0)),
            pl.BlockSpec((_RB, D), lambda i: (i, 0)),
            pl.BlockSpec((_RB, 1), lambda i: (i, 0)),
            pl.BlockSpec((D, D), lambda i: (0, 0)),
            pl.BlockSpec((D, D), lambda i: (0, 0)),
        ],
        out_specs=pl.BlockSpec((_RB, D), lambda i: (i, 0)),
        out_shape=jax.ShapeDtypeStruct((N_N, D), jnp.float32),
    )(part, part, h, ind, weight1, weight2)
    return out
